# baseline (speedup 1.0000x reference)
import numpy as np
import jax
import jax.numpy as jnp
from jax import lax
from jax.experimental import pallas as pl
from jax.experimental.pallas import tpu as pltpu
from jax.experimental.pallas import tpu_sc as plsc

_N = 100000
_E = 6400000
_H = 0.05
_GAMMA = 1.0

_LANES = 16
_NC, _NS = 2, 16
_NW = _NC * _NS
_NP = 100352
_NT = _NP // _NS
_EPW = _E // _NW
_CH = 2000
_NCHUNK = _EPW // _CH

_W1N = 20.0 * (7.0 / np.pi) / _H ** 2

_f32 = jnp.float32
_i32 = jnp.int32
_mesh = plsc.VectorSubcoreMesh(
    core_axis_name="c", subcore_axis_name="s", num_cores=_NC, num_subcores=_NS
)
_sc_params = pltpu.CompilerParams(
    use_tc_tiling_on_sc=False, needs_layout_passes=False
)


def _pass1_body(i_hbm, j_hbm, q_hbm, d2_hbm, a_hbm, rho_hbm, rho0_hbm,
                onx, ony, ocx, ocy, odeg,
                tn_sh, th_sh, anx_sh, any_sh, acx_sh, acy_sh, adeg_sh,
                nb0, nb1, nb2, nb3, nb4,
                eA, eB, ones_v, jsemA, jsemB, lsem, gsemA, gsemB, ssem):
    c = lax.axis_index("c")
    s = lax.axis_index("s")
    nb = s * _NT
    sl_n = pl.ds(nb, _NT)

    pltpu.sync_copy(a_hbm.at[sl_n], nb0)
    pltpu.sync_copy(rho_hbm.at[sl_n], nb1)
    pltpu.sync_copy(rho0_hbm.at[sl_n], nb2)
    zeros = jnp.zeros((_LANES,), _f32)

    def stage(k, _):
        sl = pl.ds(k * _LANES, _LANES)
        a = nb0[sl]
        nb3[sl] = a / nb1[sl]
        nb4[sl] = a * nb2[sl]
        nb0[sl] = zeros
        return _

    lax.fori_loop(0, _NT // _LANES, stage, None)
    pltpu.sync_copy(nb3, tn_sh.at[sl_n])
    pltpu.sync_copy(nb4, th_sh.at[sl_n])
    pltpu.sync_copy(nb0, anx_sh.at[sl_n])
    pltpu.sync_copy(nb0, any_sh.at[sl_n])
    pltpu.sync_copy(nb0, acx_sh.at[sl_n])
    pltpu.sync_copy(nb0, acy_sh.at[sl_n])
    pltpu.sync_copy(nb0, adeg_sh.at[sl_n])

    ones = jnp.ones((_LANES,), _f32)

    def fill_ones(k, _):
        ones_v[pl.ds(k * _LANES, _LANES)] = ones
        return _

    lax.fori_loop(0, _CH // _LANES, fill_ones, None)
    plsc.subcore_barrier()

    ebase = (c * _NS + s) * _EPW

    def issue_lin(base, e):
        sl_e = pl.ds(base, _CH)
        sl_d = pl.ds(pl.multiple_of(base * 2, 16), 2 * _CH)
        return [
            pltpu.async_copy(i_hbm.at[sl_e], e["iv"], lsem),
            pltpu.async_copy(q_hbm.at[sl_e], e["qv"], lsem),
            pltpu.async_copy(d2_hbm.at[sl_d], e["d2v"], lsem),
        ]

    def issue_gather(e, gsem):
        return [
            pltpu.async_copy(tn_sh.at[e["jv"]], e["fnv"], gsem),
            pltpu.async_copy(th_sh.at[e["jv"]], e["fhv"], gsem),
        ]

    def issue_scatter(e):
        return [
            pltpu.async_copy(e["bnx"], anx_sh.at[e["iv"]], ssem, add=True),
            pltpu.async_copy(e["bny"], any_sh.at[e["iv"]], ssem, add=True),
            pltpu.async_copy(e["bcx"], acx_sh.at[e["iv"]], ssem, add=True),
            pltpu.async_copy(e["bcy"], acy_sh.at[e["iv"]], ssem, add=True),
            pltpu.async_copy(ones_v, adeg_sh.at[e["iv"]], ssem, add=True),
        ]

    iota2 = lax.iota(_i32, _LANES) * 2

    def compute(e):
        def comp(k, _):
            sl = pl.ds(k * _LANES, _LANES)
            q = e["qv"][sl]
            idx = iota2 + k * (2 * _LANES)
            dx = plsc.load_gather(e["d2v"], [idx])
            dy = plsc.load_gather(e["d2v"], [idx + 1])
            t = q - 1.0
            g3 = t * t * t
            wn = (q * g3 * _W1N) * e["fnv"][sl]
            u64 = g3 * (q * q * q) * 64.0
            res = jnp.where(q <= 0.5, u64 + u64 + 1.0, u64)
            chv = (e["fhv"][sl] * _GAMMA) * res
            e["bnx"][sl] = wn * dx
            e["bny"][sl] = wn * dy
            e["bcx"][sl] = chv * dx
            e["bcy"][sl] = chv * dy
            return _

        lax.fori_loop(0, _CH // _LANES, comp, None)

    def superstep(h, _):
        baseA = pl.multiple_of(ebase + h * (2 * _CH), 8)
        baseB = pl.multiple_of(baseA + _CH, 8)
        jA = pltpu.async_copy(j_hbm.at[pl.ds(baseA, _CH)], eA["jv"], jsemA)
        jB = pltpu.async_copy(j_hbm.at[pl.ds(baseB, _CH)], eB["jv"], jsemB)
        lins = issue_lin(baseA, eA) + issue_lin(baseB, eB)
        jA.wait()
        gA = issue_gather(eA, gsemA)
        jB.wait()
        gB = issue_gather(eB, gsemB)
        for d in lins:
            d.wait()
        for d in gA:
            d.wait()
        compute(eA)
        sA = issue_scatter(eA)
        for d in gB:
            d.wait()
        compute(eB)
        sB = issue_scatter(eB)
        for d in sA + sB:
            d.wait()
        return _

    lax.fori_loop(0, _NCHUNK // 2, superstep, None)
    plsc.subcore_barrier()

    pltpu.sync_copy(anx_sh.at[sl_n], onx.at[c, sl_n])
    pltpu.sync_copy(any_sh.at[sl_n], ony.at[c, sl_n])
    pltpu.sync_copy(acx_sh.at[sl_n], ocx.at[c, sl_n])
    pltpu.sync_copy(acy_sh.at[sl_n], ocy.at[c, sl_n])
    pltpu.sync_copy(adeg_sh.at[sl_n], odeg.at[c, sl_n])


def _pass2_body(i_hbm, j_hbm, onx, ony, ocx, ocy, odeg,
                o2x, o2y,
                ngx_sh, ngy_sh, a2x_sh, a2y_sh,
                nb0, nb1, nb2, nb3, nb4,
                eA, eB, jsemA, jsemB, lsem, gsemA, gsemB, ssem):
    c = lax.axis_index("c")
    s = lax.axis_index("s")
    nb = s * _NT
    sl_n = pl.ds(nb, _NT)
    n_iters = _NT // _LANES

    pltpu.sync_copy(onx.at[0, sl_n], nb0)
    pltpu.sync_copy(onx.at[1, sl_n], nb1)
    pltpu.sync_copy(ony.at[0, sl_n], nb2)
    pltpu.sync_copy(ony.at[1, sl_n], nb3)

    def comb_n(k, _):
        sl = pl.ds(k * _LANES, _LANES)
        nb0[sl] = (nb0[sl] + nb1[sl]) * _GAMMA
        nb2[sl] = (nb2[sl] + nb3[sl]) * _GAMMA
        return _

    lax.fori_loop(0, n_iters, comb_n, None)
    pltpu.sync_copy(nb0, ngx_sh.at[sl_n])
    pltpu.sync_copy(nb2, ngy_sh.at[sl_n])

    @pl.when(c == 0)
    def _():
        pltpu.sync_copy(odeg.at[0, sl_n], nb1)
        pltpu.sync_copy(odeg.at[1, sl_n], nb3)

        def comb_d(k, _):
            sl = pl.ds(k * _LANES, _LANES)
            nb1[sl] = nb1[sl] + nb3[sl]
            return _

        lax.fori_loop(0, n_iters, comb_d, None)
        pltpu.sync_copy(ocx.at[0, sl_n], nb3)
        pltpu.sync_copy(ocx.at[1, sl_n], nb4)

        def init_x(k, _):
            sl = pl.ds(k * _LANES, _LANES)
            nb3[sl] = nb3[sl] + nb4[sl] - nb1[sl] * nb0[sl]
            return _

        lax.fori_loop(0, n_iters, init_x, None)
        pltpu.sync_copy(nb3, a2x_sh.at[sl_n])
        pltpu.sync_copy(ocy.at[0, sl_n], nb3)
        pltpu.sync_copy(ocy.at[1, sl_n], nb4)

        def init_y(k, _):
            sl = pl.ds(k * _LANES, _LANES)
            nb3[sl] = nb3[sl] + nb4[sl] - nb1[sl] * nb2[sl]
            return _

        lax.fori_loop(0, n_iters, init_y, None)
        pltpu.sync_copy(nb3, a2y_sh.at[sl_n])

    @pl.when(c == 1)
    def _():
        zeros = jnp.zeros((_LANES,), _f32)

        def zero_fill(k, _):
            nb4[pl.ds(k * _LANES, _LANES)] = zeros
            return _

        lax.fori_loop(0, n_iters, zero_fill, None)
        pltpu.sync_copy(nb4, a2x_sh.at[sl_n])
        pltpu.sync_copy(nb4, a2y_sh.at[sl_n])

    plsc.subcore_barrier()

    ebase = (c * _NS + s) * _EPW

    def superstep(h, _):
        baseA = pl.multiple_of(ebase + h * (2 * _CH), 8)
        baseB = pl.multiple_of(baseA + _CH, 8)
        jA = pltpu.async_copy(j_hbm.at[pl.ds(baseA, _CH)], eA["jv"], jsemA)
        jB = pltpu.async_copy(j_hbm.at[pl.ds(baseB, _CH)], eB["jv"], jsemB)
        iA = pltpu.async_copy(i_hbm.at[pl.ds(baseA, _CH)], eA["iv"], lsem)
        iB = pltpu.async_copy(i_hbm.at[pl.ds(baseB, _CH)], eB["iv"], lsem)
        jA.wait()
        gA = [
            pltpu.async_copy(ngx_sh.at[eA["jv"]], eA["gxv"], gsemA),
            pltpu.async_copy(ngy_sh.at[eA["jv"]], eA["gyv"], gsemA),
        ]
        jB.wait()
        gB = [
            pltpu.async_copy(ngx_sh.at[eB["jv"]], eB["gxv"], gsemB),
            pltpu.async_copy(ngy_sh.at[eB["jv"]], eB["gyv"], gsemB),
        ]
        iA.wait()
        iB.wait()
        for d in gA:
            d.wait()
        sA = [
            pltpu.async_copy(eA["gxv"], a2x_sh.at[eA["iv"]], ssem, add=True),
            pltpu.async_copy(eA["gyv"], a2y_sh.at[eA["iv"]], ssem, add=True),
        ]
        for d in gB:
            d.wait()
        sB = [
            pltpu.async_copy(eB["gxv"], a2x_sh.at[eB["iv"]], ssem, add=True),
            pltpu.async_copy(eB["gyv"], a2y_sh.at[eB["iv"]], ssem, add=True),
        ]
        for d in sA + sB:
            d.wait()
        return _

    lax.fori_loop(0, _NCHUNK // 2, superstep, None)
    plsc.subcore_barrier()
    pltpu.sync_copy(a2x_sh.at[sl_n], o2x.at[c, sl_n])
    pltpu.sync_copy(a2y_sh.at[sl_n], o2y.at[c, sl_n])


def _p1_bufs():
    d = {k: pltpu.VMEM((_CH,), _i32) for k in ("iv", "jv")}
    for k in ("qv", "fnv", "fhv", "bnx", "bny", "bcx", "bcy"):
        d[k] = pltpu.VMEM((_CH,), _f32)
    d["d2v"] = pltpu.VMEM((2 * _CH,), _f32)
    return d


def _p2_bufs():
    return {
        "iv": pltpu.VMEM((_CH,), _i32),
        "jv": pltpu.VMEM((_CH,), _i32),
        "gxv": pltpu.VMEM((_CH,), _f32),
        "gyv": pltpu.VMEM((_CH,), _f32),
    }


_pass1 = pl.kernel(
    _pass1_body,
    out_type=tuple(jax.ShapeDtypeStruct((_NC, _NP), _f32) for _ in range(5)),
    mesh=_mesh,
    scratch_types=(
        [pltpu.VMEM_SHARED((_NP,), _f32) for _ in range(7)]
        + [pltpu.VMEM((_NT,), _f32) for _ in range(5)]
        + [_p1_bufs(), _p1_bufs()]
        + [pltpu.VMEM((_CH,), _f32)]
        + [pltpu.SemaphoreType.DMA for _ in range(6)]
    ),
    compiler_params=_sc_params,
)

_pass2 = pl.kernel(
    _pass2_body,
    out_type=tuple(jax.ShapeDtypeStruct((_NC, _NP), _f32) for _ in range(2)),
    mesh=_mesh,
    scratch_types=(
        [pltpu.VMEM_SHARED((_NP,), _f32) for _ in range(4)]
        + [pltpu.VMEM((_NT,), _f32) for _ in range(5)]
        + [_p2_bufs(), _p2_bufs()]
        + [pltpu.SemaphoreType.DMA for _ in range(6)]
    ),
)


def _combine_body(x_ref, y_ref, ox_ref, oy_ref):
    ox_ref[...] = x_ref[0, :] + x_ref[1, :]
    oy_ref[...] = y_ref[0, :] + y_ref[1, :]


_combine = pl.pallas_call(
    _combine_body,
    out_shape=(
        jax.ShapeDtypeStruct((_NP,), _f32),
        jax.ShapeDtypeStruct((_NP,), _f32),
    ),
)


@jax.jit
def kernel(neighbors, fluidArea, fluidDensity, fluidRestDensity,
           fluidRadialDistances, fluidDistances):
    i = neighbors[0]
    j = neighbors[1]
    d2 = fluidDistances.reshape(-1)
    pad = _NP - _N
    ap = jnp.concatenate([fluidArea, jnp.zeros((pad,), _f32)])
    rhop = jnp.concatenate([fluidDensity, jnp.ones((pad,), _f32)])
    rho0p = jnp.concatenate([fluidRestDensity, jnp.zeros((pad,), _f32)])
    onx, ony, ocx, ocy, odeg = _pass1(
        i, j, fluidRadialDistances, d2, ap, rhop, rho0p)
    o2x, o2y = _pass2(i, j, onx, ony, ocx, ocy, odeg)
    ox, oy = _combine(o2x, o2y)
    return jnp.stack([ox[:_N], oy[:_N]], axis=-1)


# speedup vs baseline: 9.7265x; 9.7265x over previous
import numpy as np
import jax
import jax.numpy as jnp
from jax import lax
from jax.experimental import pallas as pl
from jax.experimental.pallas import tpu as pltpu
from jax.experimental.pallas import tpu_sc as plsc

_N = 100000
_E = 6400000
_H = 0.05
_GAMMA = 1.0

_LANES = 16
_NC, _NS = 2, 16
_NW = _NC * _NS
_NP = 100352
_NT = _NP // _NS
_EPW = _E // _NW
_CH = 2000
_NCHUNK = _EPW // _CH

_W1N = 20.0 * (7.0 / np.pi) / _H ** 2

_f32 = jnp.float32
_i32 = jnp.int32
_mesh = plsc.VectorSubcoreMesh(
    core_axis_name="c", subcore_axis_name="s", num_cores=_NC, num_subcores=_NS
)


def _pass1_body(i_hbm, j_hbm, q_hbm, dx_hbm, dy_hbm, a_hbm, rho_hbm, rho0_hbm,
                onx, ony, ocx, ocy, odeg,
                tn_sh, th_sh, anx_sh, any_sh, acx_sh, acy_sh, adeg_sh,
                nb0, nb1, nb2, nb3, nb4,
                eA, eB, ones_v, jsemA, jsemB, lsem, gsemA, gsemB, ssem):
    c = lax.axis_index("c")
    s = lax.axis_index("s")
    nb = s * _NT
    sl_n = pl.ds(nb, _NT)

    pltpu.sync_copy(a_hbm.at[sl_n], nb0)
    pltpu.sync_copy(rho_hbm.at[sl_n], nb1)
    pltpu.sync_copy(rho0_hbm.at[sl_n], nb2)
    zeros = jnp.zeros((_LANES,), _f32)

    def stage(k, _):
        sl = pl.ds(k * _LANES, _LANES)
        a = nb0[sl]
        nb3[sl] = a / nb1[sl]
        nb4[sl] = a * nb2[sl]
        nb0[sl] = zeros
        return _

    lax.fori_loop(0, _NT // _LANES, stage, None)
    pltpu.sync_copy(nb3, tn_sh.at[sl_n])
    pltpu.sync_copy(nb4, th_sh.at[sl_n])
    pltpu.sync_copy(nb0, anx_sh.at[sl_n])
    pltpu.sync_copy(nb0, any_sh.at[sl_n])
    pltpu.sync_copy(nb0, acx_sh.at[sl_n])
    pltpu.sync_copy(nb0, acy_sh.at[sl_n])
    pltpu.sync_copy(nb0, adeg_sh.at[sl_n])

    ones = jnp.ones((_LANES,), _f32)

    def fill_ones(k, _):
        ones_v[pl.ds(k * _LANES, _LANES)] = ones
        return _

    lax.fori_loop(0, _CH // _LANES, fill_ones, None)
    plsc.subcore_barrier()

    ebase = (c * _NS + s) * _EPW

    def issue_lin(base, e):
        sl_e = pl.ds(base, _CH)
        return [
            pltpu.async_copy(i_hbm.at[sl_e], e["iv"], lsem),
            pltpu.async_copy(q_hbm.at[sl_e], e["qv"], lsem),
            pltpu.async_copy(dx_hbm.at[sl_e], e["dxv"], lsem),
            pltpu.async_copy(dy_hbm.at[sl_e], e["dyv"], lsem),
        ]

    def issue_gather(e, gsem):
        return [
            pltpu.async_copy(tn_sh.at[e["jv"]], e["fnv"], gsem),
            pltpu.async_copy(th_sh.at[e["jv"]], e["fhv"], gsem),
        ]

    def issue_scatter(e):
        return [
            pltpu.async_copy(e["bnx"], anx_sh.at[e["iv"]], ssem, add=True),
            pltpu.async_copy(e["bny"], any_sh.at[e["iv"]], ssem, add=True),
            pltpu.async_copy(e["bcx"], acx_sh.at[e["iv"]], ssem, add=True),
            pltpu.async_copy(e["bcy"], acy_sh.at[e["iv"]], ssem, add=True),
            pltpu.async_copy(ones_v, adeg_sh.at[e["iv"]], ssem, add=True),
        ]

    def compute(e):
        def comp(k, _):
            sl = pl.ds(k * _LANES, _LANES)
            q = e["qv"][sl]
            dx = e["dxv"][sl]
            dy = e["dyv"][sl]
            t = q - 1.0
            g3 = t * t * t
            wn = (q * g3 * _W1N) * e["fnv"][sl]
            u64 = g3 * (q * q * q) * 64.0
            res = jnp.where(q <= 0.5, u64 + u64 + 1.0, u64)
            chv = (e["fhv"][sl] * _GAMMA) * res
            e["bnx"][sl] = wn * dx
            e["bny"][sl] = wn * dy
            e["bcx"][sl] = chv * dx
            e["bcy"][sl] = chv * dy
            return _

        lax.fori_loop(0, _CH // _LANES, comp, None)

    def superstep(h, _):
        baseA = pl.multiple_of(ebase + h * (2 * _CH), 8)
        baseB = pl.multiple_of(baseA + _CH, 8)
        jA = pltpu.async_copy(j_hbm.at[pl.ds(baseA, _CH)], eA["jv"], jsemA)
        jB = pltpu.async_copy(j_hbm.at[pl.ds(baseB, _CH)], eB["jv"], jsemB)
        lins = issue_lin(baseA, eA) + issue_lin(baseB, eB)
        jA.wait()
        gA = issue_gather(eA, gsemA)
        jB.wait()
        gB = issue_gather(eB, gsemB)
        for d in lins:
            d.wait()
        for d in gA:
            d.wait()
        compute(eA)
        sA = issue_scatter(eA)
        for d in gB:
            d.wait()
        compute(eB)
        sB = issue_scatter(eB)
        for d in sA + sB:
            d.wait()
        return _

    lax.fori_loop(0, _NCHUNK // 2, superstep, None)
    plsc.subcore_barrier()

    pltpu.sync_copy(anx_sh.at[sl_n], onx.at[c, sl_n])
    pltpu.sync_copy(any_sh.at[sl_n], ony.at[c, sl_n])
    pltpu.sync_copy(acx_sh.at[sl_n], ocx.at[c, sl_n])
    pltpu.sync_copy(acy_sh.at[sl_n], ocy.at[c, sl_n])
    pltpu.sync_copy(adeg_sh.at[sl_n], odeg.at[c, sl_n])


def _pass2_body(i_hbm, j_hbm, onx, ony, ocx, ocy, odeg,
                o2x, o2y,
                ngx_sh, ngy_sh, a2x_sh, a2y_sh,
                nb0, nb1, nb2, nb3, nb4,
                eA, eB, jsemA, jsemB, lsem, gsemA, gsemB, ssem):
    c = lax.axis_index("c")
    s = lax.axis_index("s")
    nb = s * _NT
    sl_n = pl.ds(nb, _NT)
    n_iters = _NT // _LANES

    pltpu.sync_copy(onx.at[0, sl_n], nb0)
    pltpu.sync_copy(onx.at[1, sl_n], nb1)
    pltpu.sync_copy(ony.at[0, sl_n], nb2)
    pltpu.sync_copy(ony.at[1, sl_n], nb3)

    def comb_n(k, _):
        sl = pl.ds(k * _LANES, _LANES)
        nb0[sl] = (nb0[sl] + nb1[sl]) * _GAMMA
        nb2[sl] = (nb2[sl] + nb3[sl]) * _GAMMA
        return _

    lax.fori_loop(0, n_iters, comb_n, None)
    pltpu.sync_copy(nb0, ngx_sh.at[sl_n])
    pltpu.sync_copy(nb2, ngy_sh.at[sl_n])

    @pl.when(c == 0)
    def _():
        pltpu.sync_copy(odeg.at[0, sl_n], nb1)
        pltpu.sync_copy(odeg.at[1, sl_n], nb3)

        def comb_d(k, _):
            sl = pl.ds(k * _LANES, _LANES)
            nb1[sl] = nb1[sl] + nb3[sl]
            return _

        lax.fori_loop(0, n_iters, comb_d, None)
        pltpu.sync_copy(ocx.at[0, sl_n], nb3)
        pltpu.sync_copy(ocx.at[1, sl_n], nb4)

        def init_x(k, _):
            sl = pl.ds(k * _LANES, _LANES)
            nb3[sl] = nb3[sl] + nb4[sl] - nb1[sl] * nb0[sl]
            return _

        lax.fori_loop(0, n_iters, init_x, None)
        pltpu.sync_copy(nb3, a2x_sh.at[sl_n])
        pltpu.sync_copy(ocy.at[0, sl_n], nb3)
        pltpu.sync_copy(ocy.at[1, sl_n], nb4)

        def init_y(k, _):
            sl = pl.ds(k * _LANES, _LANES)
            nb3[sl] = nb3[sl] + nb4[sl] - nb1[sl] * nb2[sl]
            return _

        lax.fori_loop(0, n_iters, init_y, None)
        pltpu.sync_copy(nb3, a2y_sh.at[sl_n])

    @pl.when(c == 1)
    def _():
        zeros = jnp.zeros((_LANES,), _f32)

        def zero_fill(k, _):
            nb4[pl.ds(k * _LANES, _LANES)] = zeros
            return _

        lax.fori_loop(0, n_iters, zero_fill, None)
        pltpu.sync_copy(nb4, a2x_sh.at[sl_n])
        pltpu.sync_copy(nb4, a2y_sh.at[sl_n])

    plsc.subcore_barrier()

    ebase = (c * _NS + s) * _EPW

    def superstep(h, _):
        baseA = pl.multiple_of(ebase + h * (2 * _CH), 8)
        baseB = pl.multiple_of(baseA + _CH, 8)
        jA = pltpu.async_copy(j_hbm.at[pl.ds(baseA, _CH)], eA["jv"], jsemA)
        jB = pltpu.async_copy(j_hbm.at[pl.ds(baseB, _CH)], eB["jv"], jsemB)
        iA = pltpu.async_copy(i_hbm.at[pl.ds(baseA, _CH)], eA["iv"], lsem)
        iB = pltpu.async_copy(i_hbm.at[pl.ds(baseB, _CH)], eB["iv"], lsem)
        jA.wait()
        gA = [
            pltpu.async_copy(ngx_sh.at[eA["jv"]], eA["gxv"], gsemA),
            pltpu.async_copy(ngy_sh.at[eA["jv"]], eA["gyv"], gsemA),
        ]
        jB.wait()
        gB = [
            pltpu.async_copy(ngx_sh.at[eB["jv"]], eB["gxv"], gsemB),
            pltpu.async_copy(ngy_sh.at[eB["jv"]], eB["gyv"], gsemB),
        ]
        iA.wait()
        iB.wait()
        for d in gA:
            d.wait()
        sA = [
            pltpu.async_copy(eA["gxv"], a2x_sh.at[eA["iv"]], ssem, add=True),
            pltpu.async_copy(eA["gyv"], a2y_sh.at[eA["iv"]], ssem, add=True),
        ]
        for d in gB:
            d.wait()
        sB = [
            pltpu.async_copy(eB["gxv"], a2x_sh.at[eB["iv"]], ssem, add=True),
            pltpu.async_copy(eB["gyv"], a2y_sh.at[eB["iv"]], ssem, add=True),
        ]
        for d in sA + sB:
            d.wait()
        return _

    lax.fori_loop(0, _NCHUNK // 2, superstep, None)
    plsc.subcore_barrier()
    pltpu.sync_copy(a2x_sh.at[sl_n], o2x.at[c, sl_n])
    pltpu.sync_copy(a2y_sh.at[sl_n], o2y.at[c, sl_n])


def _p1_bufs():
    d = {k: pltpu.VMEM((_CH,), _i32) for k in ("iv", "jv")}
    for k in ("qv", "dxv", "dyv", "fnv", "fhv", "bnx", "bny", "bcx", "bcy"):
        d[k] = pltpu.VMEM((_CH,), _f32)
    return d


def _p2_bufs():
    return {
        "iv": pltpu.VMEM((_CH,), _i32),
        "jv": pltpu.VMEM((_CH,), _i32),
        "gxv": pltpu.VMEM((_CH,), _f32),
        "gyv": pltpu.VMEM((_CH,), _f32),
    }


_pass1 = pl.kernel(
    _pass1_body,
    out_type=tuple(jax.ShapeDtypeStruct((_NC, _NP), _f32) for _ in range(5)),
    mesh=_mesh,
    scratch_types=(
        [pltpu.VMEM_SHARED((_NP,), _f32) for _ in range(7)]
        + [pltpu.VMEM((_NT,), _f32) for _ in range(5)]
        + [_p1_bufs(), _p1_bufs()]
        + [pltpu.VMEM((_CH,), _f32)]
        + [pltpu.SemaphoreType.DMA for _ in range(6)]
    ),
)

_pass2 = pl.kernel(
    _pass2_body,
    out_type=tuple(jax.ShapeDtypeStruct((_NC, _NP), _f32) for _ in range(2)),
    mesh=_mesh,
    scratch_types=(
        [pltpu.VMEM_SHARED((_NP,), _f32) for _ in range(4)]
        + [pltpu.VMEM((_NT,), _f32) for _ in range(5)]
        + [_p2_bufs(), _p2_bufs()]
        + [pltpu.SemaphoreType.DMA for _ in range(6)]
    ),
)


def _combine_body(x_ref, y_ref, ox_ref, oy_ref):
    ox_ref[...] = x_ref[0, :] + x_ref[1, :]
    oy_ref[...] = y_ref[0, :] + y_ref[1, :]


_combine = pl.pallas_call(
    _combine_body,
    out_shape=(
        jax.ShapeDtypeStruct((_NP,), _f32),
        jax.ShapeDtypeStruct((_NP,), _f32),
    ),
)


@jax.jit
def kernel(neighbors, fluidArea, fluidDensity, fluidRestDensity,
           fluidRadialDistances, fluidDistances):
    i = neighbors[0]
    j = neighbors[1]
    dx = fluidDistances[:, 0]
    dy = fluidDistances[:, 1]
    pad = _NP - _N
    ap = jnp.concatenate([fluidArea, jnp.zeros((pad,), _f32)])
    rhop = jnp.concatenate([fluidDensity, jnp.ones((pad,), _f32)])
    rho0p = jnp.concatenate([fluidRestDensity, jnp.zeros((pad,), _f32)])
    onx, ony, ocx, ocy, odeg = _pass1(
        i, j, fluidRadialDistances, dx, dy, ap, rhop, rho0p)
    o2x, o2y = _pass2(i, j, onx, ony, ocx, ocy, odeg)
    ox, oy = _combine(o2x, o2y)
    return jnp.stack([ox[:_N], oy[:_N]], axis=-1)
